# Initial kernel scaffold; baseline (speedup 1.0000x reference)
import jax
import jax.numpy as jnp
from jax import lax
from jax.experimental import pallas as pl
from jax.experimental.pallas import tpu as pltpu
from jax.experimental.pallas import tpu_sc as plsc

VOCAB = 100000
D = 64
B = 4096
H = 50

NC = 2
NS = 16
L = 16
NW = NC * NS
SPW = B // NW
C = 16
NCH = SPW // C
ROWS = C * H
G = 8
GSZ = ROWS // G
DB = D // L


def _body(idx_hbm, w_hbm, table_hbm, out_hbm, idx_v, rows_v, w_v, out_v, sem):
    cid = lax.axis_index("c")
    sid = lax.axis_index("s")
    wid = sid * NC + cid

    pltpu.sync_copy(w_hbm.at[wid], w_v)

    iota = lax.iota(jnp.int32, L)
    row0 = iota * H

    for c in range(NCH):
        pltpu.sync_copy(idx_hbm.at[wid, c], idx_v)
        copies = [
            pltpu.async_copy(
                table_hbm.at[idx_v.at[g]],
                rows_v.at[pl.ds(g * GSZ, GSZ)],
                sem,
            )
            for g in range(G)
        ]
        for cp in copies:
            cp.wait()

        for k in range(DB):
            def body(j, accs):
                wv = w_v[j, pl.ds(c * C, L)]
                ridx = row0 + j
                new = []
                for dd in range(L):
                    col = jnp.full((L,), k * L + dd, jnp.int32)
                    val = plsc.load_gather(rows_v, [ridx, col])
                    new.append(accs[dd] + wv * val)
                return tuple(new)

            accs = lax.fori_loop(
                0, H, body, tuple(jnp.zeros((L,), jnp.float32) for _ in range(L))
            )
            for dd in range(L):
                col = jnp.full((L,), k * L + dd, jnp.int32)
                plsc.store_scatter(out_v, [iota, col], accs[dd])

        pltpu.sync_copy(out_v, out_hbm.at[pl.ds(wid * SPW + c * C, C)])


def kernel(indices, weights, table):
    idx_r = indices.reshape(NW, NCH, G, GSZ)
    w_t = weights.reshape(NW, SPW, H).transpose(0, 2, 1)

    run = pl.kernel(
        _body,
        out_type=jax.ShapeDtypeStruct((B, D), jnp.float32),
        mesh=plsc.VectorSubcoreMesh(core_axis_name="c", subcore_axis_name="s"),
        scratch_types=[
            pltpu.VMEM((G, GSZ), jnp.int32),
            pltpu.VMEM((ROWS, D), jnp.float32),
            pltpu.VMEM((H, SPW), jnp.float32),
            pltpu.VMEM((C, D), jnp.float32),
            pltpu.SemaphoreType.DMA,
        ],
    )
    return run(idx_r, w_t, table)


# speedup vs baseline: 9.3553x; 9.3553x over previous
import jax
import jax.numpy as jnp
from jax import lax
from jax.experimental import pallas as pl
from jax.experimental.pallas import tpu as pltpu
from jax.experimental.pallas import tpu_sc as plsc

VOCAB = 100000
D = 64
B = 4096
H = 50

NC = 2
NS = 16
L = 16
NW = NC * NS
SPW = B // NW
C = 16
NCH = SPW // C
ROWS = C * H
G = 8
GSZ = ROWS // G
DB = D // L
WPW = SPW * H


def _body(idx_hbm, w_hbm, table_hbm, out_hbm,
          idx_v, rows_v, w_v, out_v, gsems, osems):
    cid = lax.axis_index("c")
    sid = lax.axis_index("s")
    wid = sid * NC + cid

    pltpu.sync_copy(w_hbm.at[wid], w_v.at[pl.ds(0, WPW)])

    def fire_gathers(c, buf):
        pltpu.sync_copy(idx_hbm.at[wid, c], idx_v.at[buf])
        return [
            pltpu.async_copy(
                table_hbm.at[idx_v.at[buf, g]],
                rows_v.at[buf, pl.ds(g * GSZ, GSZ)],
                gsems[buf],
            )
            for g in range(G)
        ]

    def compute_chunk(c, buf):
        def body(s, _):
            wb = c * ROWS + s * H
            wvs = [w_v[pl.ds(wb + L * k, L)] for k in range(DB)]
            accs = [jnp.zeros((L,), jnp.float32) for _ in range(2 * DB)]
            for j in range(H):
                w_sj = wvs[j // L][j % L]
                for k in range(DB):
                    v = rows_v[buf, s * H + j, pl.ds(L * k, L)]
                    a = (j % 2) * DB + k
                    accs[a] = accs[a] + w_sj * v
            for k in range(DB):
                out_v[buf, s, pl.ds(L * k, L)] = accs[k] + accs[DB + k]
            return 0

        lax.fori_loop(0, C, body, 0)

    pending = {0: fire_gathers(0, 0)}
    out_pending = {}
    for c in range(NCH):
        buf = c % 2
        if c + 1 < NCH:
            pending[c + 1] = fire_gathers(c + 1, (c + 1) % 2)
        for cp in pending.pop(c):
            cp.wait()
        if c - 2 in out_pending:
            out_pending.pop(c - 2).wait()
        compute_chunk(c, buf)
        out_pending[c] = pltpu.async_copy(
            out_v.at[buf],
            out_hbm.at[pl.ds(wid * SPW + c * C, C)],
            osems[buf],
        )
    for cp in out_pending.values():
        cp.wait()


def kernel(indices, weights, table):
    idx_r = indices.reshape(NW, NCH, G, GSZ)
    w_r = weights.reshape(NW, WPW)

    run = pl.kernel(
        _body,
        out_type=jax.ShapeDtypeStruct((B, D), jnp.float32),
        mesh=plsc.VectorSubcoreMesh(core_axis_name="c", subcore_axis_name="s"),
        compiler_params=pltpu.CompilerParams(
            needs_layout_passes=False, use_tc_tiling_on_sc=False
        ),
        scratch_types=[
            pltpu.VMEM((2, G, GSZ), jnp.int32),
            pltpu.VMEM((2, ROWS, D), jnp.float32),
            pltpu.VMEM((WPW + L,), jnp.float32),
            pltpu.VMEM((2, C, D), jnp.float32),
            [pltpu.SemaphoreType.DMA, pltpu.SemaphoreType.DMA],
            [pltpu.SemaphoreType.DMA, pltpu.SemaphoreType.DMA],
        ],
    )
    return run(idx_r, w_r, table)
